# baseline (speedup 1.0000x reference)
import functools

import jax
import jax.numpy as jnp
from jax import lax
from jax.experimental import pallas as pl
from jax.experimental.pallas import tpu as pltpu
from jax.experimental.pallas import tpu_sc as plsc

K = 16
N = 2048
DF = 64
PW = 16

ROWS = 256


def _knn_tc_kernel(x2_ref, x1t_ref, dist_ref, idx_ref):
    x2 = x2_ref[0]
    x1t = x1t_ref[0]
    n2 = jnp.sum(x2 * x2, axis=1, keepdims=True)
    n1 = jnp.sum(x1t * x1t, axis=0, keepdims=True)
    cross = jnp.dot(x2, x1t, preferred_element_type=jnp.float32)
    d2 = n2 + n1 - 2.0 * cross
    col = lax.broadcasted_iota(jnp.int32, (ROWS, N), 1)
    dcols = []
    icols = []
    for _ in range(K):
        m = jnp.min(d2, axis=1, keepdims=True)
        sel = jnp.min(jnp.where(d2 == m, col, N), axis=1, keepdims=True)
        dcols.append(m)
        icols.append(sel)
        d2 = jnp.where(col == sel, jnp.float32(jnp.inf), d2)
    dist_ref[0] = jnp.concatenate(dcols, axis=1)
    idx_ref[0] = jnp.concatenate(icols, axis=1)


def _knn_all_frames(x2, x1t):
    f = x2.shape[0]
    grid = (f, N // ROWS)
    return pl.pallas_call(
        _knn_tc_kernel,
        grid=grid,
        in_specs=[
            pl.BlockSpec((1, ROWS, 3), lambda i, r: (i, r, 0)),
            pl.BlockSpec((1, 3, N), lambda i, r: (i, 0, 0)),
        ],
        out_specs=[
            pl.BlockSpec((1, ROWS, K), lambda i, r: (i, r, 0)),
            pl.BlockSpec((1, ROWS, K), lambda i, r: (i, r, 0)),
        ],
        out_shape=[
            jax.ShapeDtypeStruct((f, N, K), jnp.float32),
            jax.ShapeDtypeStruct((f, N, K), jnp.int32),
        ],
    )(x2, x1t)


def _sc_chain_gather(idx_tbl, idx_col0, pts_tbl, feats_tbl, frames):
    info = plsc.get_sparse_core_info()
    nc, ns = info.num_cores, info.num_subcores
    nw = nc * ns
    spw = N // nw
    mesh = plsc.VectorSubcoreMesh(core_axis_name="c", subcore_axis_name="s")

    hf = spw * K // 2

    @functools.partial(
        pl.kernel,
        mesh=mesh,
        compiler_params=pltpu.CompilerParams(
            use_tc_tiling_on_sc=False, needs_layout_passes=False),
        out_type=(
            jax.ShapeDtypeStruct((frames, K, N), jnp.int32),
            jax.ShapeDtypeStruct((frames, 3, K, N), jnp.float32),
            jax.ShapeDtypeStruct((frames * K, DF, N), jnp.float32),
        ),
        scratch_types=[
            pltpu.VMEM((spw, K), jnp.int32),
            pltpu.VMEM((spw,), jnp.int32),
            pltpu.VMEM((spw,), jnp.int32),
            pltpu.VMEM((spw * K,), jnp.int32),
            pltpu.VMEM((K, spw), jnp.int32),
            pltpu.VMEM((hf, PW), jnp.float32),
            pltpu.VMEM((3, K, spw), jnp.float32),
            pltpu.VMEM((hf, DF), jnp.float32),
            pltpu.VMEM((K * DF, spw), jnp.float32),
            pltpu.SemaphoreType.DMA,
            pltpu.SemaphoreType.DMA,
            pltpu.SemaphoreType.DMA,
        ],
    )
    def chain_kernel(idx_hbm, col0_hbm, pts_hbm, feats_hbm,
                     patch_hbm, ppts_hbm, pfeats_hbm,
                     rows_v, c_v, craw_v, gidx_v, patchT_v,
                     pbuf_v, ptsT_v, fbuf_v, featsT_v,
                     sem, sem2, sem3):
        wid = lax.axis_index("s") * nc + lax.axis_index("c")
        wbase = wid * spw
        iota16 = lax.iota(jnp.int32, 16)
        pmask = iota16 < 3
        for f in range(frames):
            if f == 0:
                pltpu.sync_copy(idx_hbm.at[pl.ds(wbase, spw)], rows_v)
                pltpu.sync_copy(col0_hbm.at[pl.ds(wbase, spw)], craw_v)
            else:
                cp1 = pltpu.async_copy(idx_hbm.at[c_v], rows_v, sem)
                cp2 = pltpu.async_copy(col0_hbm.at[c_v], craw_v, sem2)
                cp1.wait()
                cp2.wait()

            def build_row(g, _, f=f):
                row = rows_v[g]
                gidx_v[pl.ds(g * K, K)] = row + jnp.int32(f * N)
                gf = jnp.full((16,), g, jnp.int32)
                plsc.store_scatter(patchT_v, [iota16, gf], row)
                return 0
            lax.fori_loop(0, spw, build_row, 0)
            for b2 in range(spw // 16):
                c_v[pl.ds(16 * b2, 16)] = (
                    craw_v[pl.ds(16 * b2, 16)] + jnp.int32((f + 1) * N))
            pltpu.sync_copy(
                patchT_v, patch_hbm.at[f].at[:, pl.ds(wbase, spw)])

            for h in range(2):
                gsl = gidx_v.at[pl.ds(h * hf, hf)]
                cpf = pltpu.async_copy(feats_hbm.at[gsl], fbuf_v, sem)
                cpp = pltpu.async_copy(pts_hbm.at[gsl], pbuf_v, sem3)
                cpf.wait()
                cpp.wait()

                def trans_row(g, _, h=h):
                    gi = jnp.int32(h * hf) + g
                    i = jnp.full((16,), gi // K, jnp.int32)
                    kk = gi % K
                    for c in range(DF // 16):
                        v = fbuf_v[g, pl.ds(16 * c, 16)]
                        plsc.store_scatter(
                            featsT_v,
                            [kk * DF + 16 * c + iota16, i], v)
                    pv = pbuf_v[g, pl.ds(0, 16)]
                    plsc.store_scatter(
                        ptsT_v, [iota16, jnp.full((16,), kk, jnp.int32), i],
                        pv, mask=pmask)
                    return 0
                lax.fori_loop(0, hf, trans_row, 0)

            for k in range(K):
                pltpu.sync_copy(
                    featsT_v.at[pl.ds(k * DF, DF)],
                    pfeats_hbm.at[f * K + k].at[:, pl.ds(wbase, spw)])
            pltpu.sync_copy(
                ptsT_v, ppts_hbm.at[f].at[:, :, pl.ds(wbase, spw)])

    return chain_kernel(idx_tbl, idx_col0, pts_tbl, feats_tbl)


def kernel(point_seq, feat_seq):
    b, t, n, d = point_seq.shape
    d_feat = feat_seq.shape[-1]
    frames = b * t
    x1 = point_seq.reshape(frames, n, d)
    x2 = jnp.concatenate([point_seq[:, :1], point_seq], axis=1)[:, :-1]
    x2 = x2.reshape(frames, n, d)
    x1t = x1.transpose(0, 2, 1)

    dist, idx = _knn_all_frames(x2, x1t)

    idx_tbl = idx.reshape(frames * n, K)
    idx_col0 = idx_tbl[:, 0]
    pts_tbl = jnp.pad(x1.reshape(frames * n, d), ((0, 0), (0, PW - d)))
    feats_tbl = feat_seq.reshape(frames * n, d_feat)

    patchT, pptsT, pfeatsT = _sc_chain_gather(
        idx_tbl, idx_col0, pts_tbl, feats_tbl, frames)

    patchlets = jnp.transpose(patchT.reshape(b, t, K, n), (0, 1, 3, 2))
    ppoints = jnp.transpose(pptsT.reshape(b, t, 3, K, n), (0, 1, 4, 3, 2))
    pfeats = jnp.transpose(
        pfeatsT.reshape(b, t, K, d_feat, n), (0, 1, 4, 2, 3))

    return {
        "idx": idx.reshape(b, t, n, K),
        "distances": dist.reshape(b, t, n, K),
        "patchlets": patchlets,
        "patchlet_points": ppoints,
        "patchlet_feats": pfeats,
    }


# speedup vs baseline: 1.0296x; 1.0296x over previous
import functools

import jax
import jax.numpy as jnp
from jax import lax
from jax.experimental import pallas as pl
from jax.experimental.pallas import tpu as pltpu
from jax.experimental.pallas import tpu_sc as plsc

K = 16
N = 2048
DF = 64
PW = 16

ROWS = 256


def _knn_tc_kernel(x2_ref, x1t_ref, dist_ref, idx_ref):
    x2 = x2_ref[0]
    x1t = x1t_ref[0]
    n2 = jnp.sum(x2 * x2, axis=1, keepdims=True)
    n1 = jnp.sum(x1t * x1t, axis=0, keepdims=True)
    cross = jnp.dot(x2, x1t, preferred_element_type=jnp.float32)
    d2 = n2 + n1 - 2.0 * cross
    col = lax.broadcasted_iota(jnp.int32, (ROWS, N), 1)
    dcols = []
    icols = []
    for _ in range(K):
        m = jnp.min(d2, axis=1, keepdims=True)
        sel = jnp.min(jnp.where(d2 == m, col, N), axis=1, keepdims=True)
        dcols.append(m)
        icols.append(sel)
        d2 = jnp.where(col == sel, jnp.float32(jnp.inf), d2)
    dist_ref[0] = jnp.concatenate(dcols, axis=1)
    idx_ref[0] = jnp.concatenate(icols, axis=1)


def _knn_all_frames(x2, x1t):
    f = x2.shape[0]
    grid = (f, N // ROWS)
    return pl.pallas_call(
        _knn_tc_kernel,
        grid=grid,
        in_specs=[
            pl.BlockSpec((1, ROWS, 3), lambda i, r: (i, r, 0)),
            pl.BlockSpec((1, 3, N), lambda i, r: (i, 0, 0)),
        ],
        out_specs=[
            pl.BlockSpec((1, ROWS, K), lambda i, r: (i, r, 0)),
            pl.BlockSpec((1, ROWS, K), lambda i, r: (i, r, 0)),
        ],
        out_shape=[
            jax.ShapeDtypeStruct((f, N, K), jnp.float32),
            jax.ShapeDtypeStruct((f, N, K), jnp.int32),
        ],
    )(x2, x1t)


def _sc_chain_gather(idx_tbl, idx_col0, pts_tbl, feats_tbl, frames):
    info = plsc.get_sparse_core_info()
    nc, ns = info.num_cores, info.num_subcores
    nw = nc * ns
    spw = N // nw
    mesh = plsc.VectorSubcoreMesh(core_axis_name="c", subcore_axis_name="s")

    nq = 4
    qr = spw * K // nq
    qs = spw // nq

    @functools.partial(
        pl.kernel,
        mesh=mesh,
        compiler_params=pltpu.CompilerParams(
            use_tc_tiling_on_sc=False, needs_layout_passes=False),
        out_type=(
            jax.ShapeDtypeStruct((frames, K, N), jnp.int32),
            jax.ShapeDtypeStruct((frames, 3, K, N), jnp.float32),
            jax.ShapeDtypeStruct((frames * K, DF, N), jnp.float32),
        ),
        scratch_types=[
            pltpu.VMEM((spw, K), jnp.int32),
            pltpu.VMEM((spw,), jnp.int32),
            pltpu.VMEM((spw,), jnp.int32),
            pltpu.VMEM((spw * K,), jnp.int32),
            pltpu.VMEM((K, spw), jnp.int32),
            [pltpu.VMEM((qr, PW), jnp.float32) for _ in range(2)],
            pltpu.VMEM((3, K, spw), jnp.float32),
            [pltpu.VMEM((qr, DF), jnp.float32) for _ in range(2)],
            pltpu.VMEM((K * DF, spw), jnp.float32),
            pltpu.SemaphoreType.DMA,
            pltpu.SemaphoreType.DMA,
            [pltpu.SemaphoreType.DMA for _ in range(2)],
            pltpu.SemaphoreType.DMA,
        ],
    )
    def chain_kernel(idx_hbm, col0_hbm, pts_hbm, feats_hbm,
                     patch_hbm, ppts_hbm, pfeats_hbm,
                     rows_v, c_v, craw_v, gidx_v, patchT_v,
                     pbufs, ptsT_v, fbufs, featsT_v,
                     sem, sem2, qsems, sem_out):
        wid = lax.axis_index("s") * nc + lax.axis_index("c")
        wbase = wid * spw
        iota16 = lax.iota(jnp.int32, 16)
        pmask = iota16 < 3
        rowvec = [[jnp.int32(k * DF + 16 * c) + iota16
                   for c in range(DF // 16)] for k in range(K)]
        kvec = [jnp.full((16,), k, jnp.int32) for k in range(K)]
        out_cps = []
        for f in range(frames):
            if f == 0:
                pltpu.sync_copy(idx_hbm.at[pl.ds(wbase, spw)], rows_v)
                pltpu.sync_copy(col0_hbm.at[pl.ds(wbase, spw)], craw_v)
            else:
                cp1 = pltpu.async_copy(idx_hbm.at[c_v], rows_v, sem)
                cp2 = pltpu.async_copy(col0_hbm.at[c_v], craw_v, sem2)
                cp1.wait()
                cp2.wait()

            for cp in out_cps:
                cp.wait()
            out_cps = []

            def build_row(g, _, f=f):
                row = rows_v[g]
                gidx_v[pl.ds(g * K, K)] = row + jnp.int32(f * N)
                plsc.store_scatter(
                    patchT_v, [iota16, jnp.full((16,), g, jnp.int32)], row)
                return 0
            lax.fori_loop(0, spw, build_row, 0)
            for b2 in range(spw // 16):
                c_v[pl.ds(16 * b2, 16)] = (
                    craw_v[pl.ds(16 * b2, 16)] + jnp.int32((f + 1) * N))

            cps = [None, None]

            def issue(q):
                buf = q % 2
                gsl = gidx_v.at[pl.ds(q * qr, qr)]
                cps[buf] = (
                    pltpu.async_copy(feats_hbm.at[gsl], fbufs[buf], qsems[buf]),
                    pltpu.async_copy(pts_hbm.at[gsl], pbufs[buf], qsems[buf]),
                )

            issue(0)
            for q in range(nq):
                buf = q % 2
                cps[buf][0].wait()
                cps[buf][1].wait()
                if q + 1 < nq:
                    issue(q + 1)
                fbuf_v, pbuf_v = fbufs[buf], pbufs[buf]

                def trans_row(i, _, q=q, fbuf_v=fbuf_v, pbuf_v=pbuf_v):
                    ii = jnp.full((16,), jnp.int32(q * qs) + i, jnp.int32)
                    base = i * K

                    def trans_k(k, _):
                        kv16 = jnp.full((16,), k, jnp.int32)
                        rv = k * DF + iota16
                        for c in range(DF // 16):
                            v = fbuf_v[base + k, pl.ds(16 * c, 16)]
                            plsc.store_scatter(
                                featsT_v, [rv + 16 * c, ii], v)
                        pv = pbuf_v[base + k, pl.ds(0, 16)]
                        plsc.store_scatter(
                            ptsT_v, [iota16, kv16, ii], pv, mask=pmask)
                        return 0
                    lax.fori_loop(0, K, trans_k, 0)
                    return 0
                lax.fori_loop(0, qs, trans_row, 0)

            out_cps.append(pltpu.async_copy(
                patchT_v, patch_hbm.at[f].at[:, pl.ds(wbase, spw)], sem_out))
            for k in range(K):
                out_cps.append(pltpu.async_copy(
                    featsT_v.at[pl.ds(k * DF, DF)],
                    pfeats_hbm.at[f * K + k].at[:, pl.ds(wbase, spw)],
                    sem_out))
            out_cps.append(pltpu.async_copy(
                ptsT_v, ppts_hbm.at[f].at[:, :, pl.ds(wbase, spw)], sem_out))
        for cp in out_cps:
            cp.wait()

    return chain_kernel(idx_tbl, idx_col0, pts_tbl, feats_tbl)


def kernel(point_seq, feat_seq):
    b, t, n, d = point_seq.shape
    d_feat = feat_seq.shape[-1]
    frames = b * t
    x1 = point_seq.reshape(frames, n, d)
    x2 = jnp.concatenate([point_seq[:, :1], point_seq], axis=1)[:, :-1]
    x2 = x2.reshape(frames, n, d)
    x1t = x1.transpose(0, 2, 1)

    dist, idx = _knn_all_frames(x2, x1t)

    idx_tbl = idx.reshape(frames * n, K)
    idx_col0 = idx_tbl[:, 0]
    pts_tbl = jnp.pad(x1.reshape(frames * n, d), ((0, 0), (0, PW - d)))
    feats_tbl = feat_seq.reshape(frames * n, d_feat)

    patchT, pptsT, pfeatsT = _sc_chain_gather(
        idx_tbl, idx_col0, pts_tbl, feats_tbl, frames)

    patchlets = jnp.transpose(patchT.reshape(b, t, K, n), (0, 1, 3, 2))
    ppoints = jnp.transpose(pptsT.reshape(b, t, 3, K, n), (0, 1, 4, 3, 2))
    pfeats = jnp.transpose(
        pfeatsT.reshape(b, t, K, d_feat, n), (0, 1, 4, 2, 3))

    return {
        "idx": idx.reshape(b, t, n, K),
        "distances": dist.reshape(b, t, n, K),
        "patchlets": patchlets,
        "patchlet_points": ppoints,
        "patchlet_feats": pfeats,
    }
